# Initial kernel scaffold; baseline (speedup 1.0000x reference)
import jax
import jax.numpy as jnp
from jax.experimental import pallas as pl


def kernel(feat, edge_index, edge_values, W, b, att, gamma, beta):
    raise NotImplementedError("write your pallas kernel here")


# speedup vs baseline: 4.8751x; 4.8751x over previous
import functools

import jax
import jax.numpy as jnp
from jax.experimental import pallas as pl
from jax.experimental.pallas import tpu as pltpu

N = 10000
E = 160000
DIN = 256
DOUT = 256
MULHEAD = 4
DH = 64

_BN = 1000


def _dense_body(feat_ref, w_ref, b_ref, ast_ref, h_ref, st_ref):
    x = feat_ref[...]
    h = jnp.dot(x, w_ref[...], preferred_element_type=jnp.float32) + b_ref[...]
    h = jnp.maximum(h, 0.0)
    h_ref[...] = h
    st = jnp.dot(h, ast_ref[...], preferred_element_type=jnp.float32)
    st_ref[...] = jnp.where(st >= 0, st, 0.2 * st)


def _dense_stage(feat, wcat, bcat, ast):
    grid = (N // _BN,)
    return pl.pallas_call(
        _dense_body,
        grid=grid,
        in_specs=[
            pl.BlockSpec((_BN, DIN), lambda i: (i, 0)),
            pl.BlockSpec((DIN, 2 * DOUT), lambda i: (0, 0)),
            pl.BlockSpec((1, 2 * DOUT), lambda i: (0, 0)),
            pl.BlockSpec((2 * DOUT, 128), lambda i: (0, 0)),
        ],
        out_specs=[
            pl.BlockSpec((_BN, 2 * DOUT), lambda i: (i, 0)),
            pl.BlockSpec((_BN, 128), lambda i: (i, 0)),
        ],
        out_shape=[
            jax.ShapeDtypeStruct((N, 2 * DOUT), jnp.float32),
            jax.ShapeDtypeStruct((N, 128), jnp.float32),
        ],
    )(feat, wcat, bcat, ast)


def kernel(feat, edge_index, edge_values, W, b, att, gamma, beta):
    wcat = jnp.transpose(W, (2, 0, 1, 3)).reshape(DIN, 2 * DOUT)
    bcat = b.reshape(1, 2 * DOUT)
    a_self = att[0, :, :DH]
    a_neigh = att[0, :, DH:]
    ast = jnp.zeros((2 * DOUT, 128), jnp.float32)
    for i in range(MULHEAD):
        ast = ast.at[i * DH:(i + 1) * DH, i].set(a_self[i])
        ast = ast.at[DOUT + i * DH:DOUT + (i + 1) * DH, MULHEAD + i].set(a_neigh[i])

    h, st = _dense_stage(feat, wcat, bcat, ast)
    h0 = h[:, :DOUT]
    h1 = h[:, DOUT:]
    s = st[:, :MULHEAD]
    t = st[:, MULHEAD:2 * MULHEAD]

    row = edge_index[0]
    col = edge_index[1]
    vals = (s[row] + t[col]) * edge_values[:, None]
    g = h1[col].reshape(E, MULHEAD, DH)
    contrib = (vals[:, :, None] * g).reshape(E, DOUT)
    y = jax.ops.segment_sum(contrib, row, num_segments=N)

    feat_out = jnp.concatenate([h0, y], axis=1)
    mean = feat_out.mean(axis=0)
    var = feat_out.var(axis=0)
    return (feat_out - mean) / jnp.sqrt(var + 1e-09) * gamma + beta


# speedup vs baseline: 9.1890x; 1.8849x over previous
import functools

import jax
import jax.numpy as jnp
from jax import lax
from jax.experimental import pallas as pl
from jax.experimental.pallas import tpu as pltpu
from jax.experimental.pallas import tpu_sc as plsc

N = 10000
E = 160000
DIN = 256
DOUT = 256
MULHEAD = 4
DH = 64

_BN = 1000

_NC = 2
_NS = 16
_L = 16

_NW = _NC * _NS
_EPT = E // _NW
_K = 40
_NCHUNK = _EPT // _K


def _dense_body(feat_ref, w_ref, b_ref, ast_ref, h0_ref, h1_ref, st_ref):
    x = feat_ref[...]
    h = jnp.dot(x, w_ref[...], preferred_element_type=jnp.float32) + b_ref[...]
    h = jnp.maximum(h, 0.0)
    h0_ref[...] = h[:, :DOUT]
    h1_ref[...] = h[:, DOUT:]
    st = jnp.dot(h, ast_ref[...], preferred_element_type=jnp.float32)
    st_ref[...] = jnp.where(st >= 0, st, 0.2 * st)[:, :8]


def _dense_stage(feat, wcat, bcat, ast):
    return pl.pallas_call(
        _dense_body,
        grid=(N // _BN,),
        in_specs=[
            pl.BlockSpec((_BN, DIN), lambda i: (i, 0)),
            pl.BlockSpec((DIN, 2 * DOUT), lambda i: (0, 0)),
            pl.BlockSpec((1, 2 * DOUT), lambda i: (0, 0)),
            pl.BlockSpec((2 * DOUT, 128), lambda i: (0, 0)),
        ],
        out_specs=[
            pl.BlockSpec((_BN, DOUT), lambda i: (i, 0)),
            pl.BlockSpec((_BN, DOUT), lambda i: (i, 0)),
            pl.BlockSpec((_BN, 8), lambda i: (i, 0)),
        ],
        out_shape=[
            jax.ShapeDtypeStruct((N, DOUT), jnp.float32),
            jax.ShapeDtypeStruct((N, DOUT), jnp.float32),
            jax.ShapeDtypeStruct((N, 8), jnp.float32),
        ],
    )(feat, wcat, bcat, ast)


def _sc_scale_body(h1_hbm, row_hbm, col_hbm, ev_hbm, stt_hbm, v_hbm,
                   s0, s1, s2, s3, t0, t1, t2, t3,
                   rbuf, cbuf, evbuf, a0buf, a1buf, a2buf, a3buf,
                   gbuf, sem):
    c = lax.axis_index("c")
    w = lax.axis_index("s")
    for i, tab in enumerate((s0, s1, s2, s3, t0, t1, t2, t3)):
        pltpu.sync_copy(stt_hbm.at[pl.ds(i * N, N)], tab)

    base = (w * _NC + c) * _EPT
    stabs = (s0, s1, s2, s3)
    ttabs = (t0, t1, t2, t3)
    abufs = (a0buf, a1buf, a2buf, a3buf)

    def chunk(kk, carry):
        off = base + kk * _K
        pltpu.sync_copy(row_hbm.at[pl.ds(off, _K)], rbuf)
        pltpu.sync_copy(col_hbm.at[pl.ds(off, _K)], cbuf)
        pltpu.sync_copy(ev_hbm.at[pl.ds(off, _K)], evbuf)
        gcopy = pltpu.make_async_copy(h1_hbm.at[cbuf], gbuf, sem)
        gcopy.start()
        gofs = list(range(0, _K - _L + 1, _L))
        if _K % _L:
            gofs.append(_K - _L)
        for go in gofs:
            sl = pl.ds(go, _L)
            r16 = rbuf[sl]
            c16 = cbuf[sl]
            e16 = evbuf[sl]
            for h in range(MULHEAD):
                abufs[h][sl] = (plsc.load_gather(stabs[h], [r16])
                                + plsc.load_gather(ttabs[h], [c16])) * e16
        gcopy.wait()

        def scale_e(e, carry2):
            idxv = jnp.full((_L,), e, jnp.int32)
            for h in range(MULHEAD):
                vh = plsc.load_gather(abufs[h], [idxv])
                for j in range(DH // _L):
                    slj = pl.ds(h * DH + j * _L, _L)
                    gbuf[e, slj] = gbuf[e, slj] * vh
            return carry2

        lax.fori_loop(0, _K, scale_e, 0)
        pltpu.sync_copy(gbuf, v_hbm.at[pl.ds(off, _K)])
        return carry

    lax.fori_loop(0, _NCHUNK, chunk, 0)


def _sc_scale_stage(h1, row, col, ev, stt):
    mesh = plsc.VectorSubcoreMesh(core_axis_name="c", subcore_axis_name="s")
    return pl.kernel(
        _sc_scale_body,
        out_type=jax.ShapeDtypeStruct((E, DOUT), jnp.float32),
        mesh=mesh,
        compiler_params=pltpu.CompilerParams(needs_layout_passes=False),
        scratch_types=(
            [pltpu.VMEM((N,), jnp.float32)] * 8
            + [
                pltpu.VMEM((_K,), jnp.int32),
                pltpu.VMEM((_K,), jnp.int32),
                pltpu.VMEM((_K,), jnp.float32),
                pltpu.VMEM((_K,), jnp.float32),
                pltpu.VMEM((_K,), jnp.float32),
                pltpu.VMEM((_K,), jnp.float32),
                pltpu.VMEM((_K,), jnp.float32),
                pltpu.VMEM((_K, DOUT), jnp.float32),
                pltpu.SemaphoreType.DMA,
            ]
        ),
    )(h1, row, col, ev, stt)


def _reduce_body(h0_ref, y_ref, out_ref):
    i = pl.program_id(0)
    f = jnp.concatenate([h0_ref[...], y_ref[...]], axis=1)
    ssum = jnp.sum(f, axis=0, keepdims=True)
    ssq = jnp.sum(f * f, axis=0, keepdims=True)
    acc = jnp.concatenate(
        [ssum, ssq, jnp.zeros((6, 2 * DOUT), jnp.float32)], axis=0)

    @pl.when(i == 0)
    def _():
        out_ref[...] = acc

    @pl.when(i > 0)
    def _():
        out_ref[...] = out_ref[...] + acc


def _reduce_stage(h0, y):
    return pl.pallas_call(
        _reduce_body,
        grid=(N // _BN,),
        in_specs=[
            pl.BlockSpec((_BN, DOUT), lambda i: (i, 0)),
            pl.BlockSpec((_BN, DOUT), lambda i: (i, 0)),
        ],
        out_specs=pl.BlockSpec((8, 2 * DOUT), lambda i: (0, 0)),
        out_shape=jax.ShapeDtypeStruct((8, 2 * DOUT), jnp.float32),
    )(h0, y)


def _norm_body(h0_ref, y_ref, scale_ref, shift_ref, out_ref):
    f = jnp.concatenate([h0_ref[...], y_ref[...]], axis=1)
    out_ref[...] = f * scale_ref[...] + shift_ref[...]


def _norm_stage(h0, y, scale, shift):
    return pl.pallas_call(
        _norm_body,
        grid=(N // _BN,),
        in_specs=[
            pl.BlockSpec((_BN, DOUT), lambda i: (i, 0)),
            pl.BlockSpec((_BN, DOUT), lambda i: (i, 0)),
            pl.BlockSpec((1, 2 * DOUT), lambda i: (0, 0)),
            pl.BlockSpec((1, 2 * DOUT), lambda i: (0, 0)),
        ],
        out_specs=pl.BlockSpec((_BN, 2 * DOUT), lambda i: (i, 0)),
        out_shape=jax.ShapeDtypeStruct((N, 2 * DOUT), jnp.float32),
    )(h0, y, scale, shift)


def kernel(feat, edge_index, edge_values, W, b, att, gamma, beta):
    wcat = jnp.transpose(W, (2, 0, 1, 3)).reshape(DIN, 2 * DOUT)
    bcat = b.reshape(1, 2 * DOUT)
    a_self = att[0, :, :DH]
    a_neigh = att[0, :, DH:]
    ast = jnp.zeros((2 * DOUT, 128), jnp.float32)
    for i in range(MULHEAD):
        ast = ast.at[i * DH:(i + 1) * DH, i].set(a_self[i])
        ast = ast.at[DOUT + i * DH:DOUT + (i + 1) * DH, MULHEAD + i].set(a_neigh[i])

    h0, h1, st = _dense_stage(feat, wcat, bcat, ast)
    stt = st.T.reshape(8 * N)

    row = edge_index[0]
    col = edge_index[1]
    v = _sc_scale_stage(h1, row, col, edge_values, stt)
    y = jax.ops.segment_sum(v, row, num_segments=N)

    stats = _reduce_stage(h0, y)
    ssum = stats[0]
    ssq = stats[1]
    mean = ssum / N
    var = ssq / N - mean * mean
    scale = gamma / jnp.sqrt(var + 1e-09)
    shift = beta - mean * scale
    return _norm_stage(h0, y, scale.reshape(1, -1), shift.reshape(1, -1))


# speedup vs baseline: 9.4050x; 1.0235x over previous
import functools

import jax
import jax.numpy as jnp
from jax import lax
from jax.experimental import pallas as pl
from jax.experimental.pallas import tpu as pltpu
from jax.experimental.pallas import tpu_sc as plsc

N = 10000
E = 160000
DIN = 256
DOUT = 256
MULHEAD = 4
DH = 64

_BN = 1000

_NC = 2
_NS = 16
_L = 16

_NW = _NC * _NS
_EPT = E // _NW
_K = 40
_NCHUNK = _EPT // _K


def _dense_body(feat_ref, w_ref, b_ref, ast_ref, h0_ref, h1_ref, st_ref):
    x = feat_ref[...]
    h = jnp.dot(x, w_ref[...], preferred_element_type=jnp.float32) + b_ref[...]
    h = jnp.maximum(h, 0.0)
    h0_ref[...] = h[:, :DOUT]
    h1_ref[...] = h[:, DOUT:]
    st = jnp.dot(h, ast_ref[...], preferred_element_type=jnp.float32)
    st_ref[...] = jnp.where(st >= 0, st, 0.2 * st)[:, :8]


def _dense_stage(feat, wcat, bcat, ast):
    return pl.pallas_call(
        _dense_body,
        grid=(N // _BN,),
        in_specs=[
            pl.BlockSpec((_BN, DIN), lambda i: (i, 0)),
            pl.BlockSpec((DIN, 2 * DOUT), lambda i: (0, 0)),
            pl.BlockSpec((1, 2 * DOUT), lambda i: (0, 0)),
            pl.BlockSpec((2 * DOUT, 128), lambda i: (0, 0)),
        ],
        out_specs=[
            pl.BlockSpec((_BN, DOUT), lambda i: (i, 0)),
            pl.BlockSpec((_BN, DOUT), lambda i: (i, 0)),
            pl.BlockSpec((_BN, 8), lambda i: (i, 0)),
        ],
        out_shape=[
            jax.ShapeDtypeStruct((N, DOUT), jnp.float32),
            jax.ShapeDtypeStruct((N, DOUT), jnp.float32),
            jax.ShapeDtypeStruct((N, 8), jnp.float32),
        ],
    )(feat, wcat, bcat, ast)


def _sc_scale_body(h1_hbm, row_hbm, col_hbm, ev_hbm, stt_hbm, v_hbm,
                   s0, s1, s2, s3, t0, t1, t2, t3,
                   rbuf, cbuf, evbuf, a0buf, a1buf, a2buf, a3buf,
                   gbuf, sem):
    c = lax.axis_index("c")
    w = lax.axis_index("s")
    for i, tab in enumerate((s0, s1, s2, s3, t0, t1, t2, t3)):
        pltpu.sync_copy(stt_hbm.at[pl.ds(i * N, N)], tab)

    base = (w * _NC + c) * _EPT
    stabs = (s0, s1, s2, s3)
    ttabs = (t0, t1, t2, t3)
    abufs = (a0buf, a1buf, a2buf, a3buf)

    def chunk(kk, carry):
        off = base + kk * _K
        pltpu.sync_copy(row_hbm.at[pl.ds(off, _K)], rbuf)
        pltpu.sync_copy(col_hbm.at[pl.ds(off, _K)], cbuf)
        pltpu.sync_copy(ev_hbm.at[pl.ds(off, _K)], evbuf)
        gcopy = pltpu.make_async_copy(h1_hbm.at[cbuf], gbuf, sem)
        gcopy.start()
        gofs = list(range(0, _K - _L + 1, _L))
        if _K % _L:
            gofs.append(_K - _L)
        for go in gofs:
            sl = pl.ds(go, _L)
            r16 = rbuf[sl]
            c16 = cbuf[sl]
            e16 = evbuf[sl]
            for h in range(MULHEAD):
                abufs[h][sl] = (plsc.load_gather(stabs[h], [r16])
                                + plsc.load_gather(ttabs[h], [c16])) * e16
        gcopy.wait()

        def scale_e(e, carry2):
            idxv = jnp.full((_L,), e, jnp.int32)
            for h in range(MULHEAD):
                vh = plsc.load_gather(abufs[h], [idxv])
                for j in range(DH // _L):
                    slj = pl.ds(h * DH + j * _L, _L)
                    gbuf[e, slj] = gbuf[e, slj] * vh
            return carry2

        lax.fori_loop(0, _K, scale_e, 0, unroll=8)
        pltpu.sync_copy(gbuf, v_hbm.at[pl.ds(off, _K)])
        return carry

    lax.fori_loop(0, _NCHUNK, chunk, 0)


def _sc_scale_stage(h1, row, col, ev, stt):
    mesh = plsc.VectorSubcoreMesh(core_axis_name="c", subcore_axis_name="s")
    return pl.kernel(
        _sc_scale_body,
        out_type=jax.ShapeDtypeStruct((E, DOUT), jnp.float32),
        mesh=mesh,
        compiler_params=pltpu.CompilerParams(needs_layout_passes=False),
        scratch_types=(
            [pltpu.VMEM((N,), jnp.float32)] * 8
            + [
                pltpu.VMEM((_K,), jnp.int32),
                pltpu.VMEM((_K,), jnp.int32),
                pltpu.VMEM((_K,), jnp.float32),
                pltpu.VMEM((_K,), jnp.float32),
                pltpu.VMEM((_K,), jnp.float32),
                pltpu.VMEM((_K,), jnp.float32),
                pltpu.VMEM((_K,), jnp.float32),
                pltpu.VMEM((_K, DOUT), jnp.float32),
                pltpu.SemaphoreType.DMA,
            ]
        ),
    )(h1, row, col, ev, stt)


def _reduce_body(h0_ref, y_ref, out_ref):
    i = pl.program_id(0)
    f = jnp.concatenate([h0_ref[...], y_ref[...]], axis=1)
    ssum = jnp.sum(f, axis=0, keepdims=True)
    ssq = jnp.sum(f * f, axis=0, keepdims=True)
    acc = jnp.concatenate(
        [ssum, ssq, jnp.zeros((6, 2 * DOUT), jnp.float32)], axis=0)

    @pl.when(i == 0)
    def _():
        out_ref[...] = acc

    @pl.when(i > 0)
    def _():
        out_ref[...] = out_ref[...] + acc


def _reduce_stage(h0, y):
    return pl.pallas_call(
        _reduce_body,
        grid=(N // _BN,),
        in_specs=[
            pl.BlockSpec((_BN, DOUT), lambda i: (i, 0)),
            pl.BlockSpec((_BN, DOUT), lambda i: (i, 0)),
        ],
        out_specs=pl.BlockSpec((8, 2 * DOUT), lambda i: (0, 0)),
        out_shape=jax.ShapeDtypeStruct((8, 2 * DOUT), jnp.float32),
    )(h0, y)


def _norm_body(h0_ref, y_ref, scale_ref, shift_ref, out_ref):
    f = jnp.concatenate([h0_ref[...], y_ref[...]], axis=1)
    out_ref[...] = f * scale_ref[...] + shift_ref[...]


def _norm_stage(h0, y, scale, shift):
    return pl.pallas_call(
        _norm_body,
        grid=(N // _BN,),
        in_specs=[
            pl.BlockSpec((_BN, DOUT), lambda i: (i, 0)),
            pl.BlockSpec((_BN, DOUT), lambda i: (i, 0)),
            pl.BlockSpec((1, 2 * DOUT), lambda i: (0, 0)),
            pl.BlockSpec((1, 2 * DOUT), lambda i: (0, 0)),
        ],
        out_specs=pl.BlockSpec((_BN, 2 * DOUT), lambda i: (i, 0)),
        out_shape=jax.ShapeDtypeStruct((N, 2 * DOUT), jnp.float32),
    )(h0, y, scale, shift)


def kernel(feat, edge_index, edge_values, W, b, att, gamma, beta):
    wcat = jnp.transpose(W, (2, 0, 1, 3)).reshape(DIN, 2 * DOUT)
    bcat = b.reshape(1, 2 * DOUT)
    a_self = att[0, :, :DH]
    a_neigh = att[0, :, DH:]
    ast = jnp.zeros((2 * DOUT, 128), jnp.float32)
    for i in range(MULHEAD):
        ast = ast.at[i * DH:(i + 1) * DH, i].set(a_self[i])
        ast = ast.at[DOUT + i * DH:DOUT + (i + 1) * DH, MULHEAD + i].set(a_neigh[i])

    h0, h1, st = _dense_stage(feat, wcat, bcat, ast)
    stt = st.T.reshape(8 * N)

    row = edge_index[0]
    col = edge_index[1]
    v = _sc_scale_stage(h1, row, col, edge_values, stt)
    y = jax.ops.segment_sum(v, row, num_segments=N)

    stats = _reduce_stage(h0, y)
    ssum = stats[0]
    ssq = stats[1]
    mean = ssum / N
    var = ssq / N - mean * mean
    scale = gamma / jnp.sqrt(var + 1e-09)
    shift = beta - mean * scale
    return _norm_stage(h0, y, scale.reshape(1, -1), shift.reshape(1, -1))
